# Initial kernel scaffold; baseline (speedup 1.0000x reference)
import jax
import jax.numpy as jnp
from jax.experimental import pallas as pl


def kernel(input, emb_table, linear_w):
    raise NotImplementedError("write your pallas kernel here")


# speedup vs baseline: 6.0317x; 6.0317x over previous
import functools

import jax
import jax.numpy as jnp
from jax import lax
from jax.experimental import pallas as pl
from jax.experimental.pallas import tpu as pltpu
from jax.experimental.pallas import tpu_sc as plsc

_RANK = 32
_DIM = 128


def _sc_gather(table, idx_flat, n_rows):
    info = plsc.get_sparse_core_info()
    nc, ns = info.num_cores, info.num_subcores
    nw = nc * ns
    b_per_w = n_rows // nw
    chunk = 1600
    n_chunks = b_per_w // chunk
    mesh = plsc.VectorSubcoreMesh(core_axis_name="c", subcore_axis_name="s")

    @functools.partial(
        pl.kernel,
        mesh=mesh,
        compiler_params=pltpu.CompilerParams(use_tc_tiling_on_sc=False),
        out_type=jax.ShapeDtypeStruct((n_rows, _RANK), jnp.float32),
        scratch_types=[
            pltpu.VMEM((chunk,), jnp.int32),
            pltpu.VMEM((chunk, _RANK), jnp.float32),
            pltpu.SemaphoreType.DMA,
        ],
    )
    def gather_k(table_hbm, idx_hbm, out_hbm, idx_v, rows_v, sem):
        wid = lax.axis_index("s") * nc + lax.axis_index("c")
        base = wid * b_per_w

        def body(i, carry):
            off = base + i * chunk
            pltpu.sync_copy(idx_hbm.at[pl.ds(off, chunk)], idx_v)
            pltpu.async_copy(table_hbm.at[idx_v], rows_v, sem).wait()
            pltpu.sync_copy(rows_v, out_hbm.at[pl.ds(off, chunk)])
            return carry

        lax.fori_loop(0, n_chunks, body, 0)

    return gather_k(table, idx_flat)


def _tc_project(rows, w_t, n_rows):
    blk = 4096
    grid = n_rows // blk

    def mm_k(rows_ref, w_ref, out_ref):
        out_ref[...] = jnp.dot(
            rows_ref[...], w_ref[...], preferred_element_type=jnp.float32
        )

    return pl.pallas_call(
        mm_k,
        grid=(grid,),
        in_specs=[
            pl.BlockSpec((blk, _RANK), lambda i: (i, 0)),
            pl.BlockSpec((_RANK, _DIM), lambda i: (0, 0)),
        ],
        out_specs=pl.BlockSpec((blk, _DIM), lambda i: (i, 0)),
        out_shape=jax.ShapeDtypeStruct((n_rows, _DIM), jnp.float32),
    )(rows, w_t)


def kernel(input, emb_table, linear_w):
    b, h = input.shape
    n_rows = b * h
    idx_flat = input.reshape(n_rows).astype(jnp.int32)
    rows = _sc_gather(emb_table, idx_flat, n_rows)
    out = _tc_project(rows, linear_w.T, n_rows)
    return out.reshape(b, h, _DIM)


# speedup vs baseline: 6.3360x; 1.0504x over previous
import functools

import jax
import jax.numpy as jnp
from jax import lax
from jax.experimental import pallas as pl
from jax.experimental.pallas import tpu as pltpu
from jax.experimental.pallas import tpu_sc as plsc

_RANK = 32
_DIM = 128
_PACK = _DIM // _RANK


def _sc_gather_packed(table, idx_flat, n_rows):
    info = plsc.get_sparse_core_info()
    nc, ns = info.num_cores, info.num_subcores
    nw = nc * ns
    b_per_w = n_rows // nw
    chunk = 1600
    q = chunk // _PACK
    n_chunks = b_per_w // chunk
    mesh = plsc.VectorSubcoreMesh(core_axis_name="c", subcore_axis_name="s")

    @functools.partial(
        pl.kernel,
        mesh=mesh,
        compiler_params=pltpu.CompilerParams(use_tc_tiling_on_sc=False),
        out_type=jax.ShapeDtypeStruct((n_rows // _PACK, _DIM), jnp.float32),
        scratch_types=[
            pltpu.VMEM((chunk,), jnp.int32),
            pltpu.VMEM((chunk, _RANK), jnp.float32),
            pltpu.SemaphoreType.DMA,
            pltpu.SemaphoreType.DMA,
        ],
    )
    def gather_k(table_hbm, idx_hbm, out_hbm, idx_v, rows_v, sem, sem2):
        wid = lax.axis_index("s") * nc + lax.axis_index("c")
        base = wid * b_per_w

        def body(i, carry):
            off = base + i * chunk
            pltpu.sync_copy(idx_hbm.at[pl.ds(off, chunk)], idx_v)
            copies = []
            for p in range(_PACK):
                copies.append(
                    pltpu.async_copy(
                        table_hbm.at[idx_v.at[pl.ds(p * q, q)]],
                        rows_v.at[pl.ds(p * q, q), :],
                        sem,
                    )
                )
            for c in copies:
                c.wait()
            out_base = off // _PACK
            wbs = []
            for p in range(_PACK):
                wbs.append(
                    pltpu.async_copy(
                        rows_v.at[pl.ds(p * q, q), :],
                        out_hbm.at[pl.ds(out_base, q), pl.ds(p * _RANK, _RANK)],
                        sem2,
                    )
                )
            for c in wbs:
                c.wait()
            return carry

        lax.fori_loop(0, n_chunks, body, 0)

    return gather_k(table, idx_flat)


def _tc_project(rows4, m_blockdiag, n_rows):
    q = 400
    grid = (n_rows // _PACK) // q

    def mm_k(rows_ref, m_ref, out_ref):
        y = jnp.dot(rows_ref[...], m_ref[...], preferred_element_type=jnp.float32)
        for p in range(_PACK):
            out_ref[pl.ds(p * q, q), :] = y[:, p * _DIM : (p + 1) * _DIM]

    return pl.pallas_call(
        mm_k,
        grid=(grid,),
        in_specs=[
            pl.BlockSpec((q, _DIM), lambda i: (i, 0)),
            pl.BlockSpec((_DIM, _PACK * _DIM), lambda i: (0, 0)),
        ],
        out_specs=pl.BlockSpec((_PACK * q, _DIM), lambda i: (i, 0)),
        out_shape=jax.ShapeDtypeStruct((n_rows, _DIM), jnp.float32),
    )(rows4, m_blockdiag)


def kernel(input, emb_table, linear_w):
    b, h = input.shape
    n_rows = b * h
    idx_flat = input.reshape(n_rows).astype(jnp.int32)
    rows4 = _sc_gather_packed(emb_table, idx_flat, n_rows)
    wt = linear_w.T
    m_blockdiag = jax.scipy.linalg.block_diag(*([wt] * _PACK))
    out = _tc_project(rows4, m_blockdiag, n_rows)
    return out.reshape(b, h, _DIM)


# speedup vs baseline: 8.3164x; 1.3126x over previous
import functools

import jax
import jax.numpy as jnp
from jax import lax
from jax.experimental import pallas as pl
from jax.experimental.pallas import tpu as pltpu
from jax.experimental.pallas import tpu_sc as plsc

_RANK = 32
_DIM = 128
_PACK = _DIM // _RANK


def _sc_gather_packed(table, idx_flat, n_rows):
    info = plsc.get_sparse_core_info()
    nc, ns = info.num_cores, info.num_subcores
    nw = nc * ns
    b_per_w = n_rows // nw
    chunk = 1600
    q = chunk // _PACK
    n_chunks = b_per_w // chunk
    mesh = plsc.VectorSubcoreMesh(core_axis_name="c", subcore_axis_name="s")

    @functools.partial(
        pl.kernel,
        mesh=mesh,
        compiler_params=pltpu.CompilerParams(use_tc_tiling_on_sc=False),
        out_type=jax.ShapeDtypeStruct((n_rows // _PACK, _DIM), jnp.float32),
        scratch_types=[
            pltpu.VMEM((chunk,), jnp.int32),
            pltpu.VMEM((chunk, _RANK), jnp.float32),
            pltpu.SemaphoreType.DMA,
            pltpu.SemaphoreType.DMA,
        ],
    )
    def gather_k(table_hbm, idx_hbm, out_hbm, idx_v, rows_v, sem, sem2):
        wid = lax.axis_index("s") * nc + lax.axis_index("c")
        base = wid * b_per_w

        def body(i, carry):
            off = base + i * chunk
            pltpu.sync_copy(idx_hbm.at[pl.ds(off, chunk)], idx_v)
            copies = []
            for p in range(_PACK):
                copies.append(
                    pltpu.async_copy(
                        table_hbm.at[idx_v.at[pl.ds(p * q, q)]],
                        rows_v.at[pl.ds(p * q, q), :],
                        sem,
                    )
                )
            for c in copies:
                c.wait()
            out_base = off // _PACK
            wbs = []
            for p in range(_PACK):
                wbs.append(
                    pltpu.async_copy(
                        rows_v.at[pl.ds(p * q, q), :],
                        out_hbm.at[pl.ds(out_base, q), pl.ds(p * _RANK, _RANK)],
                        sem2,
                    )
                )
            for c in wbs:
                c.wait()
            return carry

        lax.fori_loop(0, n_chunks, body, 0)

    return gather_k(table, idx_flat)


def _tc_project(rows4, m_blockdiag, n_rows):
    q = 400
    grid = (n_rows // _PACK) // q

    def mm_k(rows_ref, m_ref, out_ref):
        y = jnp.dot(rows_ref[...], m_ref[...], preferred_element_type=jnp.float32)
        for p in range(_PACK):
            out_ref[pl.ds(p * q, q), :] = y[:, p * _DIM : (p + 1) * _DIM]

    return pl.pallas_call(
        mm_k,
        grid=(grid,),
        in_specs=[
            pl.BlockSpec((q, _DIM), lambda i: (i, 0)),
            pl.BlockSpec((_DIM, _PACK * _DIM), lambda i: (0, 0)),
        ],
        out_specs=pl.BlockSpec((_PACK * q, _DIM), lambda i: (i, 0)),
        out_shape=jax.ShapeDtypeStruct((n_rows, _DIM), jnp.float32),
    )(rows4, m_blockdiag)


def kernel(input, emb_table, linear_w):
    b, h = input.shape
    n_rows = b * h
    idx_flat = input.T.reshape(n_rows).astype(jnp.int32)
    rows4 = _sc_gather_packed(emb_table, idx_flat, n_rows)
    wt = linear_w.T
    m_blockdiag = jax.scipy.linalg.block_diag(*([wt] * _PACK))
    out = _tc_project(rows4, m_blockdiag, n_rows)
    return out.reshape(h, b, _DIM).transpose(1, 0, 2)
